# baseline (speedup 1.0000x reference)
import functools

import jax
import jax.numpy as jnp
from jax import lax
from jax.experimental import pallas as pl
from jax.experimental.pallas import tpu as pltpu
from jax.experimental.pallas import tpu_sc as plsc

_PI = 3.141592653589793
_EMB_RANGE = 0.21875
_GAMMA = 12.0
_PHASE_SCALE = _PI / _EMB_RANGE

_SIN_C = (0.9999999561928109, -0.16666631903739265, 0.008332890518181353,
          -0.00019820753160373255, 2.7127955028079243e-06,
          -2.0872462480614473e-08)
_COS_C = (0.9999999922693432, -0.4999999176706864, 0.041666524297934555,
          -0.0013887970073057096, 2.47734165023444e-05,
          -2.7113293594310806e-07, 1.7368827487374006e-09)


def _poly(z, coeffs):
  acc = jnp.full_like(z, coeffs[-1])
  for c in reversed(coeffs[:-1]):
    acc = acc * z + jnp.float32(c)
  return acc


def _sqrt16(s):
  s = jnp.maximum(s, jnp.float32(1e-30))
  i = lax.bitcast_convert_type(s, jnp.int32)
  y = lax.bitcast_convert_type(
      jnp.int32(0x5F3759DF) - lax.shift_right_arithmetic(i, 1), jnp.float32)
  hs = s * jnp.float32(0.5)
  y = y * (jnp.float32(1.5) - hs * y * y)
  y = y * (jnp.float32(1.5) - hs * y * y)
  return s * y


@functools.lru_cache(maxsize=None)
def _make_sc_kernel(B, NEGN, ENT_DIM, REL_DIM, NW):
  HALF = ENT_DIM // 2
  assert REL_DIM == HALF
  assert B % NW == 0
  b_per_w = B // NW
  n_per_w = b_per_w * NEGN
  n_blocks = (NEGN + 15) // 16
  chunks = []
  off = 0
  while off < NEGN:
    c = min(128, NEGN - off)
    chunks.append((off, c))
    off += c
  mesh = plsc.VectorSubcoreMesh(core_axis_name="c", subcore_axis_name="s")
  NC = 2

  @functools.partial(
      pl.kernel,
      mesh=mesh,
      out_type=jax.ShapeDtypeStruct((B * NEGN,), jnp.float32),
      compiler_params=pltpu.CompilerParams(needs_layout_passes=False),
      scratch_types=[
          pltpu.VMEM((b_per_w,), jnp.int32),
          pltpu.VMEM((b_per_w,), jnp.int32),
          pltpu.VMEM((b_per_w, ENT_DIM), jnp.float32),
          pltpu.VMEM((b_per_w, ENT_DIM), jnp.float32),
          pltpu.VMEM((n_per_w,), jnp.int32),
          pltpu.VMEM((NEGN, ENT_DIM), jnp.float32),
          pltpu.VMEM((n_per_w,), jnp.float32),
          pltpu.SemaphoreType.DMA,
      ],
  )
  def sc_kernel(h_hbm, r_hbm, tflat_hbm, ent_hbm, rel_hbm, out_hbm,
                hidx_v, ridx_v, hrows_v, rrows_v, tidx_v, trows_v,
                score_v, sem):
    wid = lax.axis_index("s") * NC + lax.axis_index("c")
    base_b = pl.multiple_of(wid * b_per_w, 8)
    base_n = pl.multiple_of(wid * n_per_w, 8)
    lane = lax.iota(jnp.int32, 16)

    pltpu.sync_copy(h_hbm.at[pl.ds(base_b, b_per_w)], hidx_v)
    pltpu.sync_copy(r_hbm.at[pl.ds(base_b, b_per_w)], ridx_v)
    pltpu.sync_copy(tflat_hbm.at[pl.ds(base_n, n_per_w)], tidx_v)
    pltpu.async_copy(ent_hbm.at[hidx_v], hrows_v, sem).wait()
    pltpu.async_copy(rel_hbm.at[ridx_v], rrows_v, sem).wait()

    def b_body(bi, carry):
      o = pl.multiple_of(bi * NEGN, 8)
      for (coff, clen) in chunks:
        pltpu.async_copy(
            ent_hbm.at[tidx_v.at[pl.ds(pl.multiple_of(o + coff, 8), clen)]],
            trows_v.at[pl.ds(coff, clen)], sem).wait()

      rot_r = []
      rot_i = []
      for kk in range(HALF // 16):
        ph = rrows_v[bi, pl.ds(kk * 16, 16)] * jnp.float32(_PHASE_SCALE)
        z = ph * ph
        sn = ph * _poly(z, _SIN_C)
        cs = _poly(z, _COS_C)
        hr = hrows_v[bi, pl.ds(kk * 16, 16)]
        hi = hrows_v[bi, pl.ds(HALF + kk * 16, 16)]
        rot_r.append(hr * cs - hi * sn)
        rot_i.append(hr * sn + hi * cs)

      def blk_body(nb, c2):
        n0 = nb * 16
        rows = n0 + lane
        mask = rows < NEGN
        rows_c = jnp.minimum(rows, NEGN - 1)
        acc = jnp.zeros((16,), jnp.float32)
        for dd in range(HALF):
          cold = jnp.full((16,), dd, jnp.int32)
          tr = plsc.load_gather(trows_v, [rows_c, cold])
          ti = plsc.load_gather(trows_v, [rows_c, cold + HALF])
          dr = rot_r[dd // 16][dd % 16] - tr
          di = rot_i[dd // 16][dd % 16] - ti
          acc = acc + _sqrt16(dr * dr + di * di)
        plsc.store_scatter(score_v, [bi * NEGN + rows_c],
                           acc - jnp.float32(_GAMMA), mask=mask)
        return c2

      return lax.fori_loop(0, n_blocks, blk_body, carry)

    lax.fori_loop(0, b_per_w, b_body, 0)
    pltpu.sync_copy(score_v, out_hbm.at[pl.ds(base_n, n_per_w)])

  return sc_kernel


def kernel(h, r, t, batch_type, ent_emb, rel_emb):
  B, NEGN = t.shape
  info = plsc.get_sparse_core_info()
  NW = info.num_cores * info.num_subcores
  fn = _make_sc_kernel(B, NEGN, ent_emb.shape[1], rel_emb.shape[1], NW)
  rel_padded = jnp.pad(rel_emb, ((0, 0), (0, ent_emb.shape[1] - rel_emb.shape[1])))
  out = fn(h, r, t.reshape(-1), ent_emb, rel_padded)
  return out.reshape(B, NEGN)


# speedup vs baseline: 2.1855x; 2.1855x over previous
import functools

import jax
import jax.numpy as jnp
from jax import lax
from jax.experimental import pallas as pl
from jax.experimental.pallas import tpu as pltpu
from jax.experimental.pallas import tpu_sc as plsc

_PI = 3.141592653589793
_EMB_RANGE = 0.21875
_GAMMA = 12.0
_PHASE_SCALE = _PI / _EMB_RANGE


@functools.lru_cache(maxsize=None)
def _make_gather_kernel(B, NEGN, ENT_DIM, NW, CHUNK=128):
  assert B % NW == 0
  b_per_w = B // NW
  n_per_w = b_per_w * NEGN
  assert n_per_w % CHUNK == 0
  n_chunks = n_per_w // CHUNK
  assert n_chunks % 2 == 0
  mesh = plsc.VectorSubcoreMesh(core_axis_name="c", subcore_axis_name="s")
  NC = 2

  @functools.partial(
      pl.kernel,
      mesh=mesh,
      out_type=(
          jax.ShapeDtypeStruct((B, ENT_DIM), jnp.float32),
          jax.ShapeDtypeStruct((B, ENT_DIM), jnp.float32),
          jax.ShapeDtypeStruct((B * NEGN, ENT_DIM), jnp.float32),
      ),
      compiler_params=pltpu.CompilerParams(needs_layout_passes=False),
      scratch_types=[
          pltpu.VMEM((b_per_w,), jnp.int32),
          pltpu.VMEM((b_per_w,), jnp.int32),
          pltpu.VMEM((n_per_w,), jnp.int32),
          pltpu.VMEM((b_per_w, ENT_DIM), jnp.float32),
          pltpu.VMEM((b_per_w, ENT_DIM), jnp.float32),
          pltpu.VMEM((CHUNK, ENT_DIM), jnp.float32),
          pltpu.VMEM((CHUNK, ENT_DIM), jnp.float32),
          pltpu.SemaphoreType.DMA,
          pltpu.SemaphoreType.DMA,
      ],
  )
  def sc_gather(h_hbm, r_hbm, tflat_hbm, ent_hbm, rel_hbm,
                hrows_hbm, rrows_hbm, trows_hbm,
                hidx_v, ridx_v, tidx_v, hbuf_v, rbuf_v, buf0, buf1,
                sem0, sem1):
    wid = lax.axis_index("s") * NC + lax.axis_index("c")
    base_b = pl.multiple_of(wid * b_per_w, 8)
    base_n = pl.multiple_of(wid * n_per_w, 8)

    pltpu.sync_copy(h_hbm.at[pl.ds(base_b, b_per_w)], hidx_v)
    pltpu.sync_copy(r_hbm.at[pl.ds(base_b, b_per_w)], ridx_v)
    pltpu.sync_copy(tflat_hbm.at[pl.ds(base_n, n_per_w)], tidx_v)

    bufs = (buf0, buf1)
    sems = (sem0, sem1)

    def _idx(c):
      return tidx_v.at[pl.ds(pl.multiple_of(c * CHUNK, 8), CHUNK)]

    pltpu.async_copy(ent_hbm.at[_idx(0)], buf0, sem0)

    def chunk_pair(p, carry):
      c = p * 2
      for j in range(2):
        buf, sem = bufs[j], sems[j]
        nbuf, nsem = bufs[1 - j], sems[1 - j]
        cj = c + j
        pltpu.make_async_copy(ent_hbm.at[_idx(0)], buf, sem).wait()

        @pl.when(cj + 1 < n_chunks)
        def _():
          pltpu.async_copy(ent_hbm.at[_idx(cj + 1)], nbuf, nsem)

        pltpu.sync_copy(
            buf,
            trows_hbm.at[pl.ds(
                pl.multiple_of(base_n + cj * CHUNK, 8), CHUNK)])
      return carry

    lax.fori_loop(0, n_chunks // 2, chunk_pair, 0)

    pltpu.async_copy(ent_hbm.at[hidx_v], hbuf_v, sem0).wait()
    pltpu.sync_copy(hbuf_v, hrows_hbm.at[pl.ds(base_b, b_per_w)])
    pltpu.async_copy(rel_hbm.at[ridx_v], rbuf_v, sem0).wait()
    pltpu.sync_copy(rbuf_v, rrows_hbm.at[pl.ds(base_b, b_per_w)])

  return sc_gather


@functools.lru_cache(maxsize=None)
def _make_score_kernel(B, NEGN, ENT_DIM, BB=8):
  HALF = ENT_DIM // 2
  assert B % BB == 0

  def tc_score(hrows_ref, rrows_ref, t3_ref, out_ref):
    hr = hrows_ref[...]
    ph = rrows_ref[...] * jnp.float32(_PHASE_SCALE)
    sn = jnp.sin(ph)
    cs = jnp.cos(ph)
    h_re = hr[:, :HALF]
    h_im = hr[:, HALF:]
    rot = jnp.concatenate(
        [h_re * cs[:, :HALF] - h_im * sn[:, :HALF],
         h_re * sn[:, :HALF] + h_im * cs[:, :HALF]], axis=1)
    t3 = t3_ref[...]
    d = rot[:, None, :] - t3
    sq = d * d
    folded = sq + jnp.concatenate([sq[:, :, HALF:], sq[:, :, :HALF]], axis=2)
    dist = jnp.sqrt(folded)
    flat = dist.reshape(BB * NEGN, ENT_DIM)
    ones = jnp.ones((ENT_DIM, 1), jnp.float32)
    tot = jax.lax.dot(flat, ones, precision=jax.lax.Precision.HIGHEST)
    out_ref[...] = (jnp.float32(0.5) * tot.reshape(BB, NEGN)
                    - jnp.float32(_GAMMA))

  grid = (B // BB,)
  return pl.pallas_call(
      tc_score,
      grid=grid,
      in_specs=[
          pl.BlockSpec((BB, ENT_DIM), lambda i: (i, 0)),
          pl.BlockSpec((BB, ENT_DIM), lambda i: (i, 0)),
          pl.BlockSpec((BB, NEGN, ENT_DIM), lambda i: (i, 0, 0)),
      ],
      out_specs=pl.BlockSpec((BB, NEGN), lambda i: (i, 0)),
      out_shape=jax.ShapeDtypeStruct((B, NEGN), jnp.float32),
  )


def kernel(h, r, t, batch_type, ent_emb, rel_emb):
  B, NEGN = t.shape
  ENT_DIM = ent_emb.shape[1]
  info = plsc.get_sparse_core_info()
  NW = info.num_cores * info.num_subcores
  rel_padded = jnp.pad(rel_emb, ((0, 0), (0, ENT_DIM - rel_emb.shape[1])))
  gather_fn = _make_gather_kernel(B, NEGN, ENT_DIM, NW)
  hrows, rrows, trows = gather_fn(h, r, t.reshape(-1), ent_emb, rel_padded)
  score_fn = _make_score_kernel(B, NEGN, ENT_DIM)
  return score_fn(hrows, rrows, trows.reshape(B, NEGN, ENT_DIM))


# speedup vs baseline: 2.5746x; 1.1780x over previous
import functools

import jax
import jax.numpy as jnp
from jax import lax
from jax.experimental import pallas as pl
from jax.experimental.pallas import tpu as pltpu
from jax.experimental.pallas import tpu_sc as plsc

_PI = 3.141592653589793
_EMB_RANGE = 0.21875
_GAMMA = 12.0
_PHASE_SCALE = _PI / _EMB_RANGE


@functools.lru_cache(maxsize=None)
def _make_gather_kernel(B, NEGN, ENT_DIM, NW, CHUNK=128):
  assert B % NW == 0
  b_per_w = B // NW
  n_per_w = b_per_w * NEGN
  assert n_per_w % CHUNK == 0
  n_chunks = n_per_w // CHUNK
  assert n_chunks % 2 == 0
  mesh = plsc.VectorSubcoreMesh(core_axis_name="c", subcore_axis_name="s")
  NC = 2

  @functools.partial(
      pl.kernel,
      mesh=mesh,
      out_type=(
          jax.ShapeDtypeStruct((B, ENT_DIM), jnp.float32),
          jax.ShapeDtypeStruct((B, ENT_DIM), jnp.float32),
          jax.ShapeDtypeStruct((B * NEGN, ENT_DIM), jnp.float32),
      ),
      compiler_params=pltpu.CompilerParams(needs_layout_passes=False),
      scratch_types=[
          pltpu.VMEM((b_per_w,), jnp.int32),
          pltpu.VMEM((b_per_w,), jnp.int32),
          pltpu.VMEM((n_per_w,), jnp.int32),
          pltpu.VMEM((b_per_w, ENT_DIM), jnp.float32),
          pltpu.VMEM((b_per_w, ENT_DIM), jnp.float32),
          pltpu.VMEM((CHUNK, ENT_DIM), jnp.float32),
          pltpu.VMEM((CHUNK, ENT_DIM), jnp.float32),
          pltpu.SemaphoreType.DMA,
          pltpu.SemaphoreType.DMA,
      ],
  )
  def sc_gather(h_hbm, r_hbm, tflat_hbm, ent_hbm, rel_hbm,
                hrows_hbm, rrows_hbm, trows_hbm,
                hidx_v, ridx_v, tidx_v, hbuf_v, rbuf_v, buf0, buf1,
                sem0, sem1):
    wid = lax.axis_index("s") * NC + lax.axis_index("c")
    base_b = pl.multiple_of(wid * b_per_w, 8)
    base_n = pl.multiple_of(wid * n_per_w, 8)

    pltpu.sync_copy(h_hbm.at[pl.ds(base_b, b_per_w)], hidx_v)
    pltpu.sync_copy(r_hbm.at[pl.ds(base_b, b_per_w)], ridx_v)
    pltpu.sync_copy(tflat_hbm.at[pl.ds(base_n, n_per_w)], tidx_v)

    bufs = (buf0, buf1)
    sems = (sem0, sem1)

    def _idx(c):
      return tidx_v.at[pl.ds(pl.multiple_of(c * CHUNK, 8), CHUNK)]

    pltpu.async_copy(ent_hbm.at[_idx(0)], buf0, sem0)

    def chunk_pair(p, carry):
      c = p * 2
      for j in range(2):
        buf, sem = bufs[j], sems[j]
        nbuf, nsem = bufs[1 - j], sems[1 - j]
        cj = c + j
        pltpu.make_async_copy(ent_hbm.at[_idx(0)], buf, sem).wait()

        @pl.when(cj + 1 < n_chunks)
        def _():
          pltpu.async_copy(ent_hbm.at[_idx(cj + 1)], nbuf, nsem)

        pltpu.sync_copy(
            buf,
            trows_hbm.at[pl.ds(
                pl.multiple_of(base_n + cj * CHUNK, 8), CHUNK)])
      return carry

    lax.fori_loop(0, n_chunks // 2, chunk_pair, 0)

    pltpu.async_copy(ent_hbm.at[hidx_v], hbuf_v, sem0).wait()
    pltpu.sync_copy(hbuf_v, hrows_hbm.at[pl.ds(base_b, b_per_w)])
    pltpu.async_copy(rel_hbm.at[ridx_v], rbuf_v, sem0).wait()
    pltpu.sync_copy(rbuf_v, rrows_hbm.at[pl.ds(base_b, b_per_w)])

  return sc_gather


@functools.lru_cache(maxsize=None)
def _make_score_kernel(B, NEGN, ENT_DIM, BB=8, CN=8):
  HALF = ENT_DIM // 2
  assert B % BB == 0 and NEGN % CN == 0

  def tc_score(hrows_ref, rrows_ref, t3_ref, out_ref):
    hr = hrows_ref[...]
    ph = rrows_ref[...] * jnp.float32(_PHASE_SCALE)
    sn = jnp.sin(ph)
    cs = jnp.cos(ph)
    h_re = hr[:, :HALF]
    h_im = hr[:, HALF:]
    rot = jnp.concatenate(
        [h_re * cs[:, :HALF] - h_im * sn[:, :HALF],
         h_re * sn[:, :HALF] + h_im * cs[:, :HALF]], axis=1)
    rotb = rot[:, None, :]
    ones = jnp.ones((ENT_DIM, 1), jnp.float32)
    for ci in range(NEGN // CN):
      t = t3_ref[:, ci * CN:(ci + 1) * CN, :]
      d = rotb - t
      sq = d * d
      folded = sq + jnp.concatenate(
          [sq[:, :, HALF:], sq[:, :, :HALF]], axis=2)
      folded = jnp.maximum(folded, jnp.float32(1e-30))
      dist = folded * lax.rsqrt(folded)
      tot = jax.lax.dot(dist.reshape(BB * CN, ENT_DIM), ones)
      out_ref[:, ci * CN:(ci + 1) * CN] = (
          jnp.float32(0.5) * tot.reshape(BB, CN) - jnp.float32(_GAMMA))

  grid = (B // BB,)
  return pl.pallas_call(
      tc_score,
      grid=grid,
      in_specs=[
          pl.BlockSpec((BB, ENT_DIM), lambda i: (i, 0)),
          pl.BlockSpec((BB, ENT_DIM), lambda i: (i, 0)),
          pl.BlockSpec((BB, NEGN, ENT_DIM), lambda i: (i, 0, 0)),
      ],
      out_specs=pl.BlockSpec((BB, NEGN), lambda i: (i, 0)),
      out_shape=jax.ShapeDtypeStruct((B, NEGN), jnp.float32),
  )


def kernel(h, r, t, batch_type, ent_emb, rel_emb):
  B, NEGN = t.shape
  ENT_DIM = ent_emb.shape[1]
  info = plsc.get_sparse_core_info()
  NW = info.num_cores * info.num_subcores
  rel_padded = jnp.pad(rel_emb, ((0, 0), (0, ENT_DIM - rel_emb.shape[1])))
  gather_fn = _make_gather_kernel(B, NEGN, ENT_DIM, NW)
  hrows, rrows, trows = gather_fn(h, r, t.reshape(-1), ent_emb, rel_padded)
  score_fn = _make_score_kernel(B, NEGN, ENT_DIM)
  return score_fn(hrows, rrows, trows.reshape(B, NEGN, ENT_DIM))


# speedup vs baseline: 3.0407x; 1.1811x over previous
import functools

import jax
import jax.numpy as jnp
from jax import lax
from jax.experimental import pallas as pl
from jax.experimental.pallas import tpu as pltpu
from jax.experimental.pallas import tpu_sc as plsc

_PI = 3.141592653589793
_EMB_RANGE = 0.21875
_GAMMA = 12.0
_PHASE_SCALE = _PI / _EMB_RANGE


@functools.lru_cache(maxsize=None)
def _make_gather_kernel(B, NEGN, ENT_DIM, NW):
  assert B % NW == 0
  b_per_w = B // NW
  n_per_w = b_per_w * NEGN
  CHUNK = next(c for c in range(128, 0, -8)
               if n_per_w % c == 0 and (n_per_w // c) % 2 == 0)
  n_chunks = n_per_w // CHUNK
  mesh = plsc.VectorSubcoreMesh(core_axis_name="c", subcore_axis_name="s")
  NC = 2

  @functools.partial(
      pl.kernel,
      mesh=mesh,
      out_type=(
          jax.ShapeDtypeStruct((B, ENT_DIM), jnp.float32),
          jax.ShapeDtypeStruct((B, ENT_DIM), jnp.float32),
          jax.ShapeDtypeStruct((B * NEGN, ENT_DIM), jnp.float32),
      ),
      compiler_params=pltpu.CompilerParams(needs_layout_passes=False),
      scratch_types=[
          pltpu.VMEM((b_per_w,), jnp.int32),
          pltpu.VMEM((b_per_w,), jnp.int32),
          pltpu.VMEM((n_per_w,), jnp.int32),
          pltpu.VMEM((b_per_w, ENT_DIM), jnp.float32),
          pltpu.VMEM((b_per_w, ENT_DIM), jnp.float32),
          pltpu.VMEM((CHUNK, ENT_DIM), jnp.float32),
          pltpu.VMEM((CHUNK, ENT_DIM), jnp.float32),
          pltpu.SemaphoreType.DMA,
          pltpu.SemaphoreType.DMA,
      ],
  )
  def sc_gather(h_hbm, r_hbm, tflat_hbm, ent_hbm, rel_hbm,
                hrows_hbm, rrows_hbm, trows_hbm,
                hidx_v, ridx_v, tidx_v, hbuf_v, rbuf_v, buf0, buf1,
                sem0, sem1):
    wid = lax.axis_index("s") * NC + lax.axis_index("c")
    base_b = pl.multiple_of(wid * b_per_w, 8)
    base_n = pl.multiple_of(wid * n_per_w, 8)

    pltpu.sync_copy(h_hbm.at[pl.ds(base_b, b_per_w)], hidx_v)
    pltpu.sync_copy(r_hbm.at[pl.ds(base_b, b_per_w)], ridx_v)
    pltpu.sync_copy(tflat_hbm.at[pl.ds(base_n, n_per_w)], tidx_v)

    bufs = (buf0, buf1)
    sems = (sem0, sem1)

    def _idx(c):
      return tidx_v.at[pl.ds(pl.multiple_of(c * CHUNK, 8), CHUNK)]

    pltpu.async_copy(ent_hbm.at[_idx(0)], buf0, sem0)

    def chunk_pair(p, carry):
      c = p * 2
      for j in range(2):
        buf, sem = bufs[j], sems[j]
        nbuf, nsem = bufs[1 - j], sems[1 - j]
        cj = c + j
        pltpu.make_async_copy(ent_hbm.at[_idx(0)], buf, sem).wait()

        @pl.when(cj + 1 < n_chunks)
        def _():
          pltpu.async_copy(ent_hbm.at[_idx(cj + 1)], nbuf, nsem)

        pltpu.sync_copy(
            buf,
            trows_hbm.at[pl.ds(
                pl.multiple_of(base_n + cj * CHUNK, 8), CHUNK)])
      return carry

    lax.fori_loop(0, n_chunks // 2, chunk_pair, 0)

    pltpu.async_copy(ent_hbm.at[hidx_v], hbuf_v, sem0).wait()
    pltpu.sync_copy(hbuf_v, hrows_hbm.at[pl.ds(base_b, b_per_w)])
    pltpu.async_copy(rel_hbm.at[ridx_v], rbuf_v, sem0).wait()
    pltpu.sync_copy(rbuf_v, rrows_hbm.at[pl.ds(base_b, b_per_w)])

  return sc_gather


@functools.lru_cache(maxsize=None)
def _make_score_kernel(B, NEGN, ENT_DIM, BB=8, CN=8):
  HALF = ENT_DIM // 2
  assert B % BB == 0 and NEGN % CN == 0

  def tc_score(hrows_ref, rrows_ref, t3_ref, out_ref):
    hr = hrows_ref[...]
    ph = rrows_ref[...] * jnp.float32(_PHASE_SCALE)
    sn = jnp.sin(ph)
    cs = jnp.cos(ph)
    h_re = hr[:, :HALF]
    h_im = hr[:, HALF:]
    rot = jnp.concatenate(
        [h_re * cs[:, :HALF] - h_im * sn[:, :HALF],
         h_re * sn[:, :HALF] + h_im * cs[:, :HALF]], axis=1)
    rotb = rot[:, None, :]
    ones = jnp.ones((ENT_DIM, 1), jnp.float32)
    for ci in range(NEGN // CN):
      t = t3_ref[:, ci * CN:(ci + 1) * CN, :]
      d = rotb - t
      sq = d * d
      folded = sq + jnp.concatenate(
          [sq[:, :, HALF:], sq[:, :, :HALF]], axis=2)
      folded = jnp.maximum(folded, jnp.float32(1e-30))
      dist = folded * lax.rsqrt(folded)
      tot = jax.lax.dot(dist.reshape(BB * CN, ENT_DIM), ones)
      out_ref[:, ci * CN:(ci + 1) * CN] = (
          jnp.float32(0.5) * tot.reshape(BB, CN) - jnp.float32(_GAMMA))

  grid = (B // BB,)
  return pl.pallas_call(
      tc_score,
      grid=grid,
      in_specs=[
          pl.BlockSpec((BB, ENT_DIM), lambda i: (i, 0)),
          pl.BlockSpec((BB, ENT_DIM), lambda i: (i, 0)),
          pl.BlockSpec((BB, NEGN, ENT_DIM), lambda i: (i, 0, 0)),
      ],
      out_specs=pl.BlockSpec((BB, NEGN), lambda i: (i, 0)),
      out_shape=jax.ShapeDtypeStruct((B, NEGN), jnp.float32),
  )


def kernel(h, r, t, batch_type, ent_emb, rel_emb):
  B, NEGN = t.shape
  ENT_DIM = ent_emb.shape[1]
  info = plsc.get_sparse_core_info()
  NW = info.num_cores * info.num_subcores
  rel_padded = jnp.pad(rel_emb, ((0, 0), (0, ENT_DIM - rel_emb.shape[1])))
  parts = 4 if B % (4 * NW) == 0 else 1
  bp = B // parts
  gather_fn = _make_gather_kernel(bp, NEGN, ENT_DIM, NW)
  score_fn = _make_score_kernel(bp, NEGN, ENT_DIM)
  outs = []
  for p in range(parts):
    sl = slice(p * bp, (p + 1) * bp)
    hrows, rrows, trows = gather_fn(
        h[sl], r[sl], t[sl].reshape(-1), ent_emb, rel_padded)
    outs.append(score_fn(hrows, rrows, trows.reshape(bp, NEGN, ENT_DIM)))
  return jnp.concatenate(outs, axis=0) if parts > 1 else outs[0]


# speedup vs baseline: 3.0834x; 1.0140x over previous
import functools

import jax
import jax.numpy as jnp
from jax import lax
from jax.experimental import pallas as pl
from jax.experimental.pallas import tpu as pltpu
from jax.experimental.pallas import tpu_sc as plsc

_PI = 3.141592653589793
_EMB_RANGE = 0.21875
_GAMMA = 12.0
_PHASE_SCALE = _PI / _EMB_RANGE


@functools.lru_cache(maxsize=None)
def _make_gather_kernel(B, NEGN, ENT_DIM, NW):
  assert B % NW == 0
  b_per_w = B // NW
  n_per_w = b_per_w * NEGN
  NBUF = 4
  CHUNK = next(c for c in range(128, 0, -8)
               if n_per_w % c == 0 and (n_per_w // c) % NBUF == 0)
  n_chunks = n_per_w // CHUNK
  mesh = plsc.VectorSubcoreMesh(core_axis_name="c", subcore_axis_name="s")
  NC = 2

  @functools.partial(
      pl.kernel,
      mesh=mesh,
      out_type=(
          jax.ShapeDtypeStruct((B, ENT_DIM), jnp.float32),
          jax.ShapeDtypeStruct((B, ENT_DIM), jnp.float32),
          jax.ShapeDtypeStruct((B * NEGN, ENT_DIM), jnp.float32),
      ),
      compiler_params=pltpu.CompilerParams(needs_layout_passes=False),
      scratch_types=[
          pltpu.VMEM((b_per_w,), jnp.int32),
          pltpu.VMEM((b_per_w,), jnp.int32),
          pltpu.VMEM((n_per_w,), jnp.int32),
          pltpu.VMEM((b_per_w, ENT_DIM), jnp.float32),
          pltpu.VMEM((b_per_w, ENT_DIM), jnp.float32),
          pltpu.VMEM((CHUNK, ENT_DIM), jnp.float32),
          pltpu.VMEM((CHUNK, ENT_DIM), jnp.float32),
          pltpu.VMEM((CHUNK, ENT_DIM), jnp.float32),
          pltpu.VMEM((CHUNK, ENT_DIM), jnp.float32),
          pltpu.SemaphoreType.DMA,
          pltpu.SemaphoreType.DMA,
          pltpu.SemaphoreType.DMA,
          pltpu.SemaphoreType.DMA,
      ],
  )
  def sc_gather(h_hbm, r_hbm, tflat_hbm, ent_hbm, rel_hbm,
                hrows_hbm, rrows_hbm, trows_hbm,
                hidx_v, ridx_v, tidx_v, hbuf_v, rbuf_v,
                buf0, buf1, buf2, buf3, sem0, sem1, sem2, sem3):
    wid = lax.axis_index("s") * NC + lax.axis_index("c")
    base_b = pl.multiple_of(wid * b_per_w, 8)
    base_n = pl.multiple_of(wid * n_per_w, 8)

    pltpu.sync_copy(h_hbm.at[pl.ds(base_b, b_per_w)], hidx_v)
    pltpu.sync_copy(r_hbm.at[pl.ds(base_b, b_per_w)], ridx_v)
    pltpu.sync_copy(tflat_hbm.at[pl.ds(base_n, n_per_w)], tidx_v)

    bufs = (buf0, buf1, buf2, buf3)
    sems = (sem0, sem1, sem2, sem3)

    def _idx(c):
      return tidx_v.at[pl.ds(pl.multiple_of(c * CHUNK, 8), CHUNK)]

    for j in range(NBUF - 1):
      pltpu.async_copy(ent_hbm.at[_idx(j)], bufs[j], sems[j])

    def chunk_group(p, carry):
      c = p * NBUF
      for j in range(NBUF):
        buf, sem = bufs[j], sems[j]
        cj = c + j
        pltpu.make_async_copy(ent_hbm.at[_idx(0)], buf, sem).wait()
        nj = cj + NBUF - 1
        bidx = (j + NBUF - 1) % NBUF

        @pl.when(nj < n_chunks)
        def _():
          pltpu.async_copy(ent_hbm.at[_idx(nj)], bufs[bidx], sems[bidx])

        pltpu.sync_copy(
            buf,
            trows_hbm.at[pl.ds(
                pl.multiple_of(base_n + cj * CHUNK, 8), CHUNK)])
      return carry

    lax.fori_loop(0, n_chunks // NBUF, chunk_group, 0)

    pltpu.async_copy(ent_hbm.at[hidx_v], hbuf_v, sem0).wait()
    pltpu.sync_copy(hbuf_v, hrows_hbm.at[pl.ds(base_b, b_per_w)])
    pltpu.async_copy(rel_hbm.at[ridx_v], rbuf_v, sem0).wait()
    pltpu.sync_copy(rbuf_v, rrows_hbm.at[pl.ds(base_b, b_per_w)])

  return sc_gather


@functools.lru_cache(maxsize=None)
def _make_score_kernel(B, NEGN, ENT_DIM, BB=8, CN=8):
  HALF = ENT_DIM // 2
  assert B % BB == 0 and NEGN % CN == 0

  def tc_score(hrows_ref, rrows_ref, t3_ref, out_ref):
    hr = hrows_ref[...]
    ph = rrows_ref[...] * jnp.float32(_PHASE_SCALE)
    sn = jnp.sin(ph)
    cs = jnp.cos(ph)
    h_re = hr[:, :HALF]
    h_im = hr[:, HALF:]
    rot = jnp.concatenate(
        [h_re * cs[:, :HALF] - h_im * sn[:, :HALF],
         h_re * sn[:, :HALF] + h_im * cs[:, :HALF]], axis=1)
    rotb = rot[:, None, :]
    ones = jnp.ones((ENT_DIM, 1), jnp.float32)
    for ci in range(NEGN // CN):
      t = t3_ref[:, ci * CN:(ci + 1) * CN, :]
      d = rotb - t
      sq = d * d
      folded = sq + jnp.concatenate(
          [sq[:, :, HALF:], sq[:, :, :HALF]], axis=2)
      folded = folded + jnp.float32(1e-30)
      dist = folded * lax.rsqrt(folded)
      tot = jax.lax.dot(dist.reshape(BB * CN, ENT_DIM), ones)
      out_ref[:, ci * CN:(ci + 1) * CN] = (
          jnp.float32(0.5) * tot.reshape(BB, CN) - jnp.float32(_GAMMA))

  grid = (B // BB,)
  return pl.pallas_call(
      tc_score,
      grid=grid,
      in_specs=[
          pl.BlockSpec((BB, ENT_DIM), lambda i: (i, 0)),
          pl.BlockSpec((BB, ENT_DIM), lambda i: (i, 0)),
          pl.BlockSpec((BB, NEGN, ENT_DIM), lambda i: (i, 0, 0)),
      ],
      out_specs=pl.BlockSpec((BB, NEGN), lambda i: (i, 0)),
      out_shape=jax.ShapeDtypeStruct((B, NEGN), jnp.float32),
  )


def kernel(h, r, t, batch_type, ent_emb, rel_emb):
  B, NEGN = t.shape
  ENT_DIM = ent_emb.shape[1]
  info = plsc.get_sparse_core_info()
  NW = info.num_cores * info.num_subcores
  rel_padded = jnp.pad(rel_emb, ((0, 0), (0, ENT_DIM - rel_emb.shape[1])))
  parts = 4 if B % (4 * NW * 8) == 0 else 1
  bp = B // parts
  gather_fn = _make_gather_kernel(bp, NEGN, ENT_DIM, NW)
  score_fn = _make_score_kernel(bp, NEGN, ENT_DIM)
  outs = []
  for p in range(parts):
    sl = slice(p * bp, (p + 1) * bp)
    hrows, rrows, trows = gather_fn(
        h[sl], r[sl], t[sl].reshape(-1), ent_emb, rel_padded)
    outs.append(score_fn(hrows, rrows, trows.reshape(bp, NEGN, ENT_DIM)))
  return jnp.concatenate(outs, axis=0) if parts > 1 else outs[0]


# speedup vs baseline: 3.7681x; 1.2221x over previous
import functools

import jax
import jax.numpy as jnp
from jax import lax
from jax.experimental import pallas as pl
from jax.experimental.pallas import tpu as pltpu
from jax.experimental.pallas import tpu_sc as plsc

_PI = 3.141592653589793
_EMB_RANGE = 0.21875
_GAMMA = 12.0
_PHASE_SCALE = _PI / _EMB_RANGE


@functools.lru_cache(maxsize=None)
def _make_gather_kernel(B, NEGN, ENT_DIM, NW):
  assert B % NW == 0
  b_per_w = B // NW
  n_per_w = b_per_w * NEGN
  NBUF = 4
  CHUNK = next(c for c in range(128, 0, -8)
               if n_per_w % c == 0 and (n_per_w // c) % NBUF == 0)
  n_chunks = n_per_w // CHUNK
  mesh = plsc.VectorSubcoreMesh(core_axis_name="c", subcore_axis_name="s")
  NC = 2

  @functools.partial(
      pl.kernel,
      mesh=mesh,
      out_type=(
          jax.ShapeDtypeStruct((B, ENT_DIM), jnp.float32),
          jax.ShapeDtypeStruct((B, ENT_DIM), jnp.float32),
          jax.ShapeDtypeStruct((B * NEGN, ENT_DIM), jnp.float32),
      ),
      compiler_params=pltpu.CompilerParams(needs_layout_passes=False),
      scratch_types=[
          pltpu.VMEM((b_per_w,), jnp.int32),
          pltpu.VMEM((b_per_w,), jnp.int32),
          pltpu.VMEM((n_per_w,), jnp.int32),
          pltpu.VMEM((b_per_w, ENT_DIM), jnp.float32),
          pltpu.VMEM((b_per_w, ENT_DIM), jnp.float32),
          pltpu.VMEM((CHUNK, ENT_DIM), jnp.float32),
          pltpu.VMEM((CHUNK, ENT_DIM), jnp.float32),
          pltpu.VMEM((CHUNK, ENT_DIM), jnp.float32),
          pltpu.VMEM((CHUNK, ENT_DIM), jnp.float32),
          pltpu.SemaphoreType.DMA,
          pltpu.SemaphoreType.DMA,
          pltpu.SemaphoreType.DMA,
          pltpu.SemaphoreType.DMA,
      ],
  )
  def sc_gather(h_hbm, r_hbm, tflat_hbm, ent_hbm, rel_hbm,
                hrows_hbm, rrows_hbm, trows_hbm,
                hidx_v, ridx_v, tidx_v, hbuf_v, rbuf_v,
                buf0, buf1, buf2, buf3, sem0, sem1, sem2, sem3):
    wid = lax.axis_index("s") * NC + lax.axis_index("c")
    base_b = pl.multiple_of(wid * b_per_w, 8)
    base_n = pl.multiple_of(wid * n_per_w, 8)

    pltpu.sync_copy(h_hbm.at[pl.ds(base_b, b_per_w)], hidx_v)
    pltpu.sync_copy(r_hbm.at[pl.ds(base_b, b_per_w)], ridx_v)
    pltpu.sync_copy(tflat_hbm.at[pl.ds(base_n, n_per_w)], tidx_v)

    bufs = (buf0, buf1, buf2, buf3)
    sems = (sem0, sem1, sem2, sem3)

    def _idx(c):
      return tidx_v.at[pl.ds(pl.multiple_of(c * CHUNK, 8), CHUNK)]

    for j in range(NBUF - 1):
      pltpu.async_copy(ent_hbm.at[_idx(j)], bufs[j], sems[j])

    def chunk_group(p, carry):
      c = p * NBUF
      for j in range(NBUF):
        buf, sem = bufs[j], sems[j]
        cj = c + j
        pltpu.make_async_copy(ent_hbm.at[_idx(0)], buf, sem).wait()
        nj = cj + NBUF - 1
        bidx = (j + NBUF - 1) % NBUF

        @pl.when(nj < n_chunks)
        def _():
          pltpu.async_copy(ent_hbm.at[_idx(nj)], bufs[bidx], sems[bidx])

        pltpu.sync_copy(
            buf,
            trows_hbm.at[pl.ds(
                pl.multiple_of(base_n + cj * CHUNK, 8), CHUNK)])
      return carry

    lax.fori_loop(0, n_chunks // NBUF, chunk_group, 0)

    pltpu.async_copy(ent_hbm.at[hidx_v], hbuf_v, sem0).wait()
    pltpu.sync_copy(hbuf_v, hrows_hbm.at[pl.ds(base_b, b_per_w)])
    pltpu.async_copy(rel_hbm.at[ridx_v], rbuf_v, sem0).wait()
    pltpu.sync_copy(rbuf_v, rrows_hbm.at[pl.ds(base_b, b_per_w)])

  return sc_gather


@functools.lru_cache(maxsize=None)
def _make_score_kernel(B, NEGN, ENT_DIM, BB=16, CN=8):
  HALF = ENT_DIM // 2
  assert B % BB == 0 and NEGN % CN == 0

  SB = 8

  def tc_score(hrows_ref, rrows_ref, t3_ref, out_ref):
    hr = hrows_ref[...]
    ph = rrows_ref[...] * jnp.float32(_PHASE_SCALE)
    sn = jnp.sin(ph)
    cs = jnp.cos(ph)
    h_re = hr[:, :HALF]
    h_im = hr[:, HALF:]
    rot = jnp.concatenate(
        [h_re * cs[:, :HALF] - h_im * sn[:, :HALF],
         h_re * sn[:, :HALF] + h_im * cs[:, :HALF]], axis=1)
    ones = jnp.ones((ENT_DIM, 1), jnp.float32)
    for bj in range(BB // SB):
      rotb = rot[bj * SB:(bj + 1) * SB, None, :]
      for ci in range(NEGN // CN):
        t = t3_ref[bj * SB:(bj + 1) * SB, ci * CN:(ci + 1) * CN, :]
        d = rotb - t
        sq = d * d
        folded = sq + jnp.concatenate(
            [sq[:, :, HALF:], sq[:, :, :HALF]], axis=2)
        folded = folded + jnp.float32(1e-30)
        dist = folded * lax.rsqrt(folded)
        tot = jax.lax.dot(dist.reshape(SB * CN, ENT_DIM), ones)
        out_ref[bj * SB:(bj + 1) * SB, ci * CN:(ci + 1) * CN] = (
            jnp.float32(0.5) * tot.reshape(SB, CN) - jnp.float32(_GAMMA))

  grid = (B // BB,)
  return pl.pallas_call(
      tc_score,
      grid=grid,
      in_specs=[
          pl.BlockSpec((BB, ENT_DIM), lambda i: (i, 0)),
          pl.BlockSpec((BB, ENT_DIM), lambda i: (i, 0)),
          pl.BlockSpec((BB, NEGN, ENT_DIM), lambda i: (i, 0, 0)),
      ],
      out_specs=pl.BlockSpec((BB, NEGN), lambda i: (i, 0)),
      out_shape=jax.ShapeDtypeStruct((B, NEGN), jnp.float32),
  )


def kernel(h, r, t, batch_type, ent_emb, rel_emb):
  B, NEGN = t.shape
  ENT_DIM = ent_emb.shape[1]
  info = plsc.get_sparse_core_info()
  NW = info.num_cores * info.num_subcores
  rel_padded = jnp.pad(rel_emb, ((0, 0), (0, ENT_DIM - rel_emb.shape[1])))
  parts = 4 if B % (4 * NW * 8) == 0 else 1
  bp = B // parts
  gather_fn = _make_gather_kernel(bp, NEGN, ENT_DIM, NW)
  score_fn = _make_score_kernel(bp, NEGN, ENT_DIM)
  outs = []
  for p in range(parts):
    sl = slice(p * bp, (p + 1) * bp)
    hrows, rrows, trows = gather_fn(
        h[sl], r[sl], t[sl].reshape(-1), ent_emb, rel_padded)
    outs.append(score_fn(hrows, rrows, trows.reshape(bp, NEGN, ENT_DIM)))
  return jnp.concatenate(outs, axis=0) if parts > 1 else outs[0]


# speedup vs baseline: 3.7916x; 1.0062x over previous
import functools

import jax
import jax.numpy as jnp
from jax import lax
from jax.experimental import pallas as pl
from jax.experimental.pallas import tpu as pltpu
from jax.experimental.pallas import tpu_sc as plsc

_PI = 3.141592653589793
_EMB_RANGE = 0.21875
_GAMMA = 12.0
_PHASE_SCALE = _PI / _EMB_RANGE


@functools.lru_cache(maxsize=None)
def _make_gather_kernel(B, NEGN, ENT_DIM, NW):
  assert B % NW == 0
  b_per_w = B // NW
  n_per_w = b_per_w * NEGN
  NBUF = 4
  CHUNK = next(c for c in range(128, 0, -8)
               if n_per_w % c == 0 and (n_per_w // c) % NBUF == 0)
  n_chunks = n_per_w // CHUNK
  mesh = plsc.VectorSubcoreMesh(core_axis_name="c", subcore_axis_name="s")
  NC = 2

  @functools.partial(
      pl.kernel,
      mesh=mesh,
      out_type=(
          jax.ShapeDtypeStruct((B, ENT_DIM), jnp.float32),
          jax.ShapeDtypeStruct((B, ENT_DIM), jnp.float32),
          jax.ShapeDtypeStruct((B * NEGN, ENT_DIM), jnp.float32),
      ),
      compiler_params=pltpu.CompilerParams(needs_layout_passes=False),
      scratch_types=[
          pltpu.VMEM((b_per_w,), jnp.int32),
          pltpu.VMEM((b_per_w,), jnp.int32),
          pltpu.VMEM((n_per_w,), jnp.int32),
          pltpu.VMEM((b_per_w, ENT_DIM), jnp.float32),
          pltpu.VMEM((b_per_w, ENT_DIM), jnp.float32),
          pltpu.VMEM((CHUNK, ENT_DIM), jnp.float32),
          pltpu.VMEM((CHUNK, ENT_DIM), jnp.float32),
          pltpu.VMEM((CHUNK, ENT_DIM), jnp.float32),
          pltpu.VMEM((CHUNK, ENT_DIM), jnp.float32),
          pltpu.SemaphoreType.DMA,
          pltpu.SemaphoreType.DMA,
          pltpu.SemaphoreType.DMA,
          pltpu.SemaphoreType.DMA,
          pltpu.SemaphoreType.DMA,
          pltpu.SemaphoreType.DMA,
          pltpu.SemaphoreType.DMA,
          pltpu.SemaphoreType.DMA,
          pltpu.SemaphoreType.DMA,
      ],
  )
  def sc_gather(h_hbm, r_hbm, tflat_hbm, ent_hbm, rel_hbm,
                hrows_hbm, rrows_hbm, trows_hbm,
                hidx_v, ridx_v, tidx_v, hbuf_v, rbuf_v,
                buf0, buf1, buf2, buf3,
                sem0, sem1, sem2, sem3,
                wsem0, wsem1, wsem2, wsem3, hrsem):
    wid = lax.axis_index("s") * NC + lax.axis_index("c")
    base_b = pl.multiple_of(wid * b_per_w, 8)
    base_n = pl.multiple_of(wid * n_per_w, 8)

    pltpu.sync_copy(h_hbm.at[pl.ds(base_b, b_per_w)], hidx_v)
    pltpu.sync_copy(r_hbm.at[pl.ds(base_b, b_per_w)], ridx_v)
    pltpu.sync_copy(tflat_hbm.at[pl.ds(base_n, n_per_w)], tidx_v)

    bufs = (buf0, buf1, buf2, buf3)
    sems = (sem0, sem1, sem2, sem3)
    wsems = (wsem0, wsem1, wsem2, wsem3)

    def _idx(c):
      return tidx_v.at[pl.ds(pl.multiple_of(c * CHUNK, 8), CHUNK)]

    def _out(c):
      return trows_hbm.at[pl.ds(pl.multiple_of(base_n + c * CHUNK, 8), CHUNK)]

    for j in range(NBUF - 1):
      pltpu.async_copy(ent_hbm.at[_idx(j)], bufs[j], sems[j])
    pltpu.async_copy(ent_hbm.at[hidx_v], hbuf_v, hrsem)
    pltpu.async_copy(rel_hbm.at[ridx_v], rbuf_v, hrsem)

    def chunk_group(p, carry):
      c = p * NBUF
      for j in range(NBUF):
        cj = c + j
        bidx = (j + NBUF - 1) % NBUF
        pltpu.make_async_copy(ent_hbm.at[_idx(0)], bufs[j], sems[j]).wait()

        @pl.when(cj >= 1)
        def _():
          pltpu.make_async_copy(bufs[bidx], _out(0), wsems[bidx]).wait()

        nj = cj + NBUF - 1

        @pl.when(nj < n_chunks)
        def _():
          pltpu.async_copy(ent_hbm.at[_idx(nj)], bufs[bidx], sems[bidx])

        pltpu.async_copy(bufs[j], _out(cj), wsems[j])
      return carry

    lax.fori_loop(0, n_chunks // NBUF, chunk_group, 0)
    lastb = (n_chunks - 1) % NBUF
    pltpu.make_async_copy(bufs[lastb], _out(0), wsems[lastb]).wait()

    pltpu.make_async_copy(ent_hbm.at[hidx_v], hbuf_v, hrsem).wait()
    pltpu.make_async_copy(rel_hbm.at[ridx_v], rbuf_v, hrsem).wait()
    pltpu.sync_copy(hbuf_v, hrows_hbm.at[pl.ds(base_b, b_per_w)])
    pltpu.sync_copy(rbuf_v, rrows_hbm.at[pl.ds(base_b, b_per_w)])

  return sc_gather


@functools.lru_cache(maxsize=None)
def _make_score_kernel(B, NEGN, ENT_DIM, BB=16, CN=8):
  HALF = ENT_DIM // 2
  assert B % BB == 0 and NEGN % CN == 0

  SB = 8

  def tc_score(hrows_ref, rrows_ref, t3_ref, out_ref):
    hr = hrows_ref[...]
    ph = rrows_ref[...] * jnp.float32(_PHASE_SCALE)
    sn = jnp.sin(ph)
    cs = jnp.cos(ph)
    h_re = hr[:, :HALF]
    h_im = hr[:, HALF:]
    rot = jnp.concatenate(
        [h_re * cs[:, :HALF] - h_im * sn[:, :HALF],
         h_re * sn[:, :HALF] + h_im * cs[:, :HALF]], axis=1)
    ones = jnp.ones((ENT_DIM, 1), jnp.float32)
    for bj in range(BB // SB):
      rotb = rot[bj * SB:(bj + 1) * SB, None, :]
      for ci in range(NEGN // CN):
        t = t3_ref[bj * SB:(bj + 1) * SB, ci * CN:(ci + 1) * CN, :]
        d = rotb - t
        sq = d * d
        folded = sq + jnp.concatenate(
            [sq[:, :, HALF:], sq[:, :, :HALF]], axis=2)
        folded = folded + jnp.float32(1e-30)
        dist = folded * lax.rsqrt(folded)
        tot = jax.lax.dot(dist.reshape(SB * CN, ENT_DIM), ones)
        out_ref[bj * SB:(bj + 1) * SB, ci * CN:(ci + 1) * CN] = (
            jnp.float32(0.5) * tot.reshape(SB, CN) - jnp.float32(_GAMMA))

  grid = (B // BB,)
  return pl.pallas_call(
      tc_score,
      grid=grid,
      in_specs=[
          pl.BlockSpec((BB, ENT_DIM), lambda i: (i, 0)),
          pl.BlockSpec((BB, ENT_DIM), lambda i: (i, 0)),
          pl.BlockSpec((BB, NEGN, ENT_DIM), lambda i: (i, 0, 0)),
      ],
      out_specs=pl.BlockSpec((BB, NEGN), lambda i: (i, 0)),
      out_shape=jax.ShapeDtypeStruct((B, NEGN), jnp.float32),
  )


def kernel(h, r, t, batch_type, ent_emb, rel_emb):
  B, NEGN = t.shape
  ENT_DIM = ent_emb.shape[1]
  info = plsc.get_sparse_core_info()
  NW = info.num_cores * info.num_subcores
  rel_padded = jnp.pad(rel_emb, ((0, 0), (0, ENT_DIM - rel_emb.shape[1])))
  parts = 4 if B % (4 * NW * 8) == 0 else 1
  bp = B // parts
  gather_fn = _make_gather_kernel(bp, NEGN, ENT_DIM, NW)
  score_fn = _make_score_kernel(bp, NEGN, ENT_DIM)
  outs = []
  for p in range(parts):
    sl = slice(p * bp, (p + 1) * bp)
    hrows, rrows, trows = gather_fn(
        h[sl], r[sl], t[sl].reshape(-1), ent_emb, rel_padded)
    outs.append(score_fn(hrows, rrows, trows.reshape(bp, NEGN, ENT_DIM)))
  return jnp.concatenate(outs, axis=0) if parts > 1 else outs[0]
